# baseline (speedup 1.0000x reference)
import functools

import jax
import jax.numpy as jnp
from jax import lax
from jax.experimental import pallas as pl
from jax.experimental.pallas import tpu as pltpu
from jax.experimental.pallas import tpu_sc as plsc

_NC = 2
_NS = 16

_CH = 128


def _sc_aggregate(x0, x1, src2, dst2, zeros, nchunk):
    n, dh = x0.shape
    base_chunks = nchunk // _NS
    extra = nchunk - base_chunks * _NS
    maxrows = base_chunks + (1 if extra else 0)
    rows_per_tile = n // _NS
    didx_rows = 80
    pairs_a = didx_rows // 2
    pairs_b = (maxrows + 1) // 2 + 1 - pairs_a

    mesh = plsc.VectorSubcoreMesh(core_axis_name="c", subcore_axis_name="s")

    @functools.partial(
        pl.kernel,
        out_type=jax.ShapeDtypeStruct((_NC, n, dh), jnp.float32),
        mesh=mesh,
        scratch_types=[
            pltpu.VMEM((maxrows, _CH), jnp.int32),
            pltpu.VMEM((didx_rows, _CH), jnp.int32),
            pltpu.VMEM((2, _CH, dh), jnp.float32),
            pltpu.VMEM_SHARED((n, dh), jnp.float32),
            pltpu.VMEM_SHARED((n, dh), jnp.float32),
            pltpu.SemaphoreType.DMA,
        ],
        compiler_params=pltpu.CompilerParams(use_tc_tiling_on_sc=False),
    )
    def agg(x0_hbm, x1_hbm, src_hbm, dst_hbm, z_hbm, out_hbm,
            sidx, didx, rows, acc_sh, xs_sh, gsem):
        c = lax.axis_index("c")
        s = lax.axis_index("s")
        base = s * base_chunks + jnp.minimum(s, extra)
        cnt = base_chunks + (s < extra).astype(jnp.int32)
        rslice = pl.ds(s * rows_per_tile, rows_per_tile)
        pltpu.sync_copy(z_hbm, acc_sh.at[rslice])

        @pl.when(c == 0)
        def _():
            pltpu.sync_copy(x0_hbm.at[rslice], xs_sh.at[rslice])

        @pl.when(c == 1)
        def _():
            pltpu.sync_copy(x1_hbm.at[rslice], xs_sh.at[rslice])

        pltpu.sync_copy(src_hbm.at[pl.ds(base, maxrows)], sidx)
        pltpu.sync_copy(dst_hbm.at[pl.ds(base, didx_rows)], didx)
        plsc.subcore_barrier()

        def gather(k, buf):
            pltpu.async_copy(xs_sh.at[sidx.at[k]], rows.at[buf], gsem)

        def gather_wait(k, buf):
            pltpu.make_async_copy(xs_sh.at[sidx.at[k]], rows.at[buf], gsem).wait()

        def make_pair(off):
            def pair(k2, carry):
                for b in range(2):
                    k = k2 * 2 + b
                    nxt = k + 1

                    @pl.when(nxt < cnt)
                    def _():
                        gather(nxt, 1 - b)

                    @pl.when(k < cnt)
                    def _():
                        gather_wait(k, b)
                        pltpu.sync_copy(rows.at[b], acc_sh.at[didx.at[k - off]],
                                        add=True)

                return carry

            return pair

        gather(0, 0)
        lax.fori_loop(0, pairs_a, make_pair(0), 0)
        pltpu.sync_copy(dst_hbm.at[pl.ds(base + didx_rows, didx_rows)], didx)
        lax.fori_loop(pairs_a, pairs_a + pairs_b, make_pair(didx_rows), 0)
        plsc.subcore_barrier()
        pltpu.sync_copy(
            acc_sh.at[pl.ds(s * rows_per_tile, rows_per_tile)],
            out_hbm.at[c, pl.ds(s * rows_per_tile, rows_per_tile)],
        )

    return agg(x0, x1, src2, dst2, zeros)


def _tc_tail(partials, x, w_rel, w_root, b_rel):
    n, d = x.shape
    dh = d // 2
    bn = 2000
    grid = (n // bn,)

    def body(p_ref, x_ref, wr_ref, wt_ref, b_ref, o_ref):
        aggr = jnp.concatenate([p_ref[0], p_ref[1]], axis=-1)
        acc = lax.dot_general(aggr, wr_ref[...], (((1,), (1,)), ((), ())),
                              preferred_element_type=jnp.float32)
        acc += lax.dot_general(x_ref[...], wt_ref[...], (((1,), (1,)), ((), ())),
                               preferred_element_type=jnp.float32)
        o_ref[...] = jnp.maximum(acc + b_ref[...], 0.0)

    return pl.pallas_call(
        body,
        grid=grid,
        in_specs=[
            pl.BlockSpec((_NC, bn, dh), lambda i: (0, i, 0)),
            pl.BlockSpec((bn, d), lambda i: (i, 0)),
            pl.BlockSpec((d, d), lambda i: (0, 0)),
            pl.BlockSpec((d, d), lambda i: (0, 0)),
            pl.BlockSpec((1, d), lambda i: (0, 0)),
        ],
        out_specs=pl.BlockSpec((bn, d), lambda i: (i, 0)),
        out_shape=jax.ShapeDtypeStruct((n, d), jnp.float32),
    )(partials, x, w_rel, w_root, b_rel)


def kernel(x, edge_index, W_rel, b_rel, W_root):
    n, d = x.shape
    dh = d // 2
    e = edge_index.shape[1]
    nchunk = e // _CH
    nchunk_pad = nchunk + 8
    pad = nchunk_pad * _CH - e
    ei = jnp.concatenate(
        [edge_index, jnp.zeros((2, pad), jnp.int32)], axis=1
    ).reshape(2, nchunk_pad, _CH)
    x0 = x[:, :dh]
    x1 = x[:, dh:]
    zeros = jnp.zeros((n // _NS, dh), jnp.float32)
    partials = _sc_aggregate(x0, x1, ei[0], ei[1], zeros, nchunk)
    return _tc_tail(partials, x, W_rel, W_root, b_rel.reshape(1, d))


# speedup vs baseline: 1.5013x; 1.5013x over previous
import functools

import jax
import jax.numpy as jnp
from jax import lax
from jax.experimental import pallas as pl
from jax.experimental.pallas import tpu as pltpu
from jax.experimental.pallas import tpu_sc as plsc

_NC = 2
_NS = 16
_NW = _NC * _NS

_CH = 128
_SPAN = 80
_DPH = 40


def _sc_aggregate(x, src2, dst2, zeros, nchunk):
    n, d = x.shape
    rows_a = (n // _NS) // 8 * 8
    rows_last = n - rows_a * (_NS - 1)

    mesh = plsc.VectorSubcoreMesh(core_axis_name="c", subcore_axis_name="s")

    @functools.partial(
        pl.kernel,
        out_type=jax.ShapeDtypeStruct((_NC, n, d), jnp.float32),
        mesh=mesh,
        scratch_types=[
            pltpu.VMEM((_SPAN, _CH), jnp.int32),
            pltpu.VMEM((_DPH, _CH), jnp.int32),
            pltpu.VMEM((2, _CH, d), jnp.float32),
            pltpu.VMEM_SHARED((n, d), jnp.float32),
            pltpu.SemaphoreType.DMA,
            pltpu.SemaphoreType.DMA,
        ],
    )
    def agg(x_hbm, src_hbm, dst_hbm, z_hbm, out_hbm,
            sidx, didx, rows, acc_sh, gsem, ssem):
        c = lax.axis_index("c")
        s = lax.axis_index("s")
        w = s * _NC + c
        base = w * _SPAN
        cnt = jnp.clip(nchunk - base, 0, _SPAN)
        @pl.when(s < _NS - 1)
        def _():
            pltpu.sync_copy(z_hbm.at[pl.ds(0, rows_a)],
                            acc_sh.at[pl.ds(s * rows_a, rows_a)])

        @pl.when(s == _NS - 1)
        def _():
            pltpu.sync_copy(z_hbm, acc_sh.at[pl.ds((_NS - 1) * rows_a, rows_last)])

        pltpu.sync_copy(src_hbm.at[pl.ds(base, _SPAN)], sidx)
        pltpu.sync_copy(dst_hbm.at[pl.ds(base, _DPH)], didx)
        plsc.subcore_barrier()

        def gather(k, buf):
            pltpu.async_copy(x_hbm.at[sidx.at[k]], rows.at[buf], gsem)

        def gather_wait(k, buf):
            pltpu.make_async_copy(x_hbm.at[sidx.at[k]], rows.at[buf], gsem).wait()

        def scatter_wait():
            pltpu.make_async_copy(rows.at[0], acc_sh.at[didx.at[0]], ssem).wait()

        def make_pair(off, k_start):
            def pair(k2, carry):
                for b in range(2):
                    k = k2 * 2 + b
                    nxt = k + 1

                    @pl.when((k > k_start) & (k - 1 < cnt))
                    def _():
                        scatter_wait()

                    @pl.when(nxt < cnt)
                    def _():
                        gather(nxt, 1 - b)

                    @pl.when(k < cnt)
                    def _():
                        gather_wait(k, b)
                        pltpu.async_copy(rows.at[b], acc_sh.at[didx.at[k - off]],
                                         ssem, add=True)

                return carry

            return pair

        gather(0, 0)
        lax.fori_loop(0, _DPH // 2, make_pair(0, 0), 0)
        @pl.when(cnt >= _DPH)
        def _():
            scatter_wait()

        pltpu.sync_copy(dst_hbm.at[pl.ds(base + _DPH, _DPH)], didx)
        lax.fori_loop(_DPH // 2, _SPAN // 2 + 2, make_pair(_DPH, _DPH), 0)
        plsc.subcore_barrier()

        @pl.when(s < _NS - 1)
        def _():
            pltpu.sync_copy(acc_sh.at[pl.ds(s * rows_a, rows_a)],
                            out_hbm.at[c, pl.ds(s * rows_a, rows_a)])

        @pl.when(s == _NS - 1)
        def _():
            pltpu.sync_copy(acc_sh.at[pl.ds((_NS - 1) * rows_a, rows_last)],
                            out_hbm.at[c, pl.ds((_NS - 1) * rows_a, rows_last)])

    return agg(x, src2, dst2, zeros)


def _tc_tail(partials, x, w_rel, w_root, b_rel):
    n, d = x.shape
    bn = 2000
    grid = (n // bn,)

    def body(p_ref, x_ref, wr_ref, wt_ref, b_ref, o_ref):
        aggr = p_ref[0] + p_ref[1]
        acc = lax.dot_general(aggr, wr_ref[...], (((1,), (1,)), ((), ())),
                              preferred_element_type=jnp.float32)
        acc += lax.dot_general(x_ref[...], wt_ref[...], (((1,), (1,)), ((), ())),
                               preferred_element_type=jnp.float32)
        o_ref[...] = jnp.maximum(acc + b_ref[...], 0.0)

    return pl.pallas_call(
        body,
        grid=grid,
        in_specs=[
            pl.BlockSpec((_NC, bn, d), lambda i: (0, i, 0)),
            pl.BlockSpec((bn, d), lambda i: (i, 0)),
            pl.BlockSpec((d, d), lambda i: (0, 0)),
            pl.BlockSpec((d, d), lambda i: (0, 0)),
            pl.BlockSpec((1, d), lambda i: (0, 0)),
        ],
        out_specs=pl.BlockSpec((bn, d), lambda i: (i, 0)),
        out_shape=jax.ShapeDtypeStruct((n, d), jnp.float32),
    )(partials, x, w_rel, w_root, b_rel)


def kernel(x, edge_index, W_rel, b_rel, W_root):
    n, d = x.shape
    e = edge_index.shape[1]
    nchunk = e // _CH
    pad = _NW * _SPAN * _CH - e
    ei = jnp.concatenate(
        [edge_index, jnp.zeros((2, pad), jnp.int32)], axis=1
    ).reshape(2, _NW * _SPAN, _CH)
    rows_last = n - (n // _NS) // 8 * 8 * (_NS - 1)
    zeros = jnp.zeros((rows_last, d), jnp.float32)
    partials = _sc_aggregate(x, ei[0], ei[1], zeros, nchunk)
    return _tc_tail(partials, x, W_rel, W_root, b_rel.reshape(1, d))
